# baseline (speedup 1.0000x reference)
import jax
import jax.numpy as jnp
from jax import lax
from jax.experimental import pallas as pl
from jax.experimental.pallas import tpu as pltpu
from jax.experimental.pallas import tpu_sc as plsc

B, C, H, W = 8, 3, 512, 512
MAX_DISP = 32.0
NC, NS = 2, 16
NW = NC * NS
W_PER_B = NW // B
ROWS_PER_W = H // W_PER_B
RBLK = 8
NBLK = ROWS_PER_W // RBLK
NPAIR = NBLK // 2
NCH = W // 16


def _round_half_even_i32(x):
    xph = x + 0.5
    f = xph.astype(jnp.int32)
    is_half = f.astype(jnp.float32) == xph
    odd = jnp.bitwise_and(f, 1)
    return f - jnp.where(is_half, odd, 0)


def _body(image_hbm, depth_hbm, out_l_hbm, out_r_hbm,
          depth_v, img_v, cnt_v, acc_v, out_v,
          in_sem0, in_sem1, out_sem0, out_sem1):
    wid = lax.axis_index("s") * NC + lax.axis_index("c")
    b = wid // W_PER_B
    y_base = (wid % W_PER_B) * ROWS_PER_W
    in_sems = (in_sem0, in_sem1)
    out_sems = (out_sem0, out_sem1)
    out_hbms = (out_l_hbm, out_r_hbm)

    xiota = lax.iota(jnp.int32, 16)
    ones = jnp.ones((16,), jnp.float32)
    zeros = jnp.zeros((16,), jnp.float32)

    def zc(k, c2):
        cnt_v[pl.ds(k * 16, 16)] = zeros
        return c2
    lax.fori_loop(0, 2 * RBLK * W // 16, zc, 0)

    def za(k, c2):
        acc_v[pl.ds(k * 16, 16)] = zeros
        return c2
    lax.fori_loop(0, 6 * RBLK * W // 16, za, 0)

    def issue_in(blk, buf):
        ys = y_base + blk * RBLK
        d1 = pltpu.async_copy(depth_hbm.at[b, pl.ds(ys, RBLK), :],
                              depth_v.at[buf], in_sems[buf])
        d2 = pltpu.async_copy(image_hbm.at[b, :, pl.ds(ys, RBLK), :],
                              img_v.at[buf], in_sems[buf])
        return d1, d2

    def wait_in(buf):
        pltpu.make_async_copy(depth_hbm.at[0, pl.ds(0, RBLK), :],
                              depth_v.at[buf], in_sems[buf]).wait()
        pltpu.make_async_copy(image_hbm.at[0, :, pl.ds(0, RBLK), :],
                              img_v.at[buf], in_sems[buf]).wait()

    def issue_out(blk, buf):
        ys = y_base + blk * RBLK
        for di in range(2):
            pltpu.async_copy(out_v.at[buf, di],
                             out_hbms[di].at[b, :, pl.ds(ys, RBLK), :],
                             out_sems[buf])

    def wait_out(buf):
        for di in range(2):
            pltpu.make_async_copy(out_hbms[di].at[0, :, pl.ds(0, RBLK), :],
                                  out_v.at[buf, di], out_sems[buf]).wait()

    def compute_block(buf):
        def srow(r, c1):
            cb = [jnp.broadcast_to((di * RBLK + r) * W, (16,)).astype(jnp.int32)
                  for di in range(2)]
            ab = [[jnp.broadcast_to(((di * C + c) * RBLK + r) * W, (16,))
                   .astype(jnp.int32) for c in range(C)] for di in range(2)]

            def sj(j, c2):
                xo = j * 16
                d16 = depth_v[buf, r, pl.ds(xo, 16)]
                disp = _round_half_even_i32(d16 * MAX_DISP)
                xb = xiota + xo
                vals = [img_v[buf, c, r, pl.ds(xo, 16)] for c in range(C)]
                for di in range(2):
                    xt = xb - disp if di == 0 else xb + disp
                    msk = (xt >= 0) & (xt < W)
                    xtc = jnp.clip(xt, 0, W - 1)
                    plsc.addupdate_scatter(cnt_v, [cb[di] + xtc], ones,
                                           mask=msk)
                    for c in range(C):
                        plsc.addupdate_scatter(acc_v, [ab[di][c] + xtc],
                                               vals[c], mask=msk)
                return c2
            lax.fori_loop(0, NCH, sj, 0)
            return c1
        lax.fori_loop(0, RBLK, srow, 0)

        def frow(r, c1):
            def fj(j, c2):
                xo = j * 16
                for di in range(2):
                    co = (di * RBLK + r) * W + xo
                    cnt = cnt_v[pl.ds(co, 16)]
                    inv = 1.0 / jnp.maximum(cnt, 1.0)
                    cnt_v[pl.ds(co, 16)] = zeros
                    for c in range(C):
                        ao = ((di * C + c) * RBLK + r) * W + xo
                        a = acc_v[pl.ds(ao, 16)]
                        out_v[buf, di, c, r, pl.ds(xo, 16)] = (
                            jnp.clip(a * inv, 0.0, 1.0))
                        acc_v[pl.ds(ao, 16)] = zeros
                return c2
            lax.fori_loop(0, NCH, fj, 0)
            return c1
        lax.fori_loop(0, RBLK, frow, 0)

    issue_in(0, 0)

    def pair(t, carry):
        blk0 = 2 * t
        blk1 = blk0 + 1
        issue_in(blk1, 1)
        wait_in(0)

        @pl.when(t > 0)
        def _():
            wait_out(0)
        compute_block(0)
        issue_out(blk0, 0)

        @pl.when(t < NPAIR - 1)
        def _():
            issue_in(blk0 + 2, 0)
        wait_in(1)

        @pl.when(t > 0)
        def _():
            wait_out(1)
        compute_block(1)
        issue_out(blk1, 1)
        return carry

    lax.fori_loop(0, NPAIR, pair, 0)
    wait_out(0)
    wait_out(1)


def kernel(image, depth):
    mesh = plsc.VectorSubcoreMesh(core_axis_name="c", subcore_axis_name="s",
                                  num_cores=NC, num_subcores=NS)
    f = pl.kernel(
        _body,
        out_type=(jax.ShapeDtypeStruct((B, C, H, W), jnp.float32),
                  jax.ShapeDtypeStruct((B, C, H, W), jnp.float32)),
        mesh=mesh,
        scratch_types=[
            pltpu.VMEM((2, RBLK, W), jnp.float32),
            pltpu.VMEM((2, C, RBLK, W), jnp.float32),
            pltpu.VMEM((2 * RBLK * W,), jnp.float32),
            pltpu.VMEM((2 * C * RBLK * W,), jnp.float32),
            pltpu.VMEM((2, 2, C, RBLK, W), jnp.float32),
            pltpu.SemaphoreType.DMA,
            pltpu.SemaphoreType.DMA,
            pltpu.SemaphoreType.DMA,
            pltpu.SemaphoreType.DMA,
        ],
        compiler_params=pltpu.CompilerParams(use_tc_tiling_on_sc=False,
                                             needs_layout_passes=False),
    )
    return f(image, depth)
